# Initial kernel scaffold; baseline (speedup 1.0000x reference)
import jax
import jax.numpy as jnp
from jax.experimental import pallas as pl


def kernel(x, W, edge_index):
    raise NotImplementedError("write your pallas kernel here")


# speedup vs baseline: 9.3473x; 9.3473x over previous
import functools
import jax
import jax.numpy as jnp
from jax import lax
from jax.experimental import pallas as pl
from jax.experimental.pallas import tpu as pltpu
import jax.experimental.pallas.tpu_sc as plsc

N = 10000
E = 320000
D = 128
K = 4
NC = 2
NS = 16
NW = NC * NS
EPW = E // NW
NCH = 125
CH = EPW // NCH
RPT = N // NS

_MESH = plsc.VectorSubcoreMesh(core_axis_name="c", subcore_axis_name="s",
                               num_cores=NC, num_subcores=NS)


@functools.partial(
    pl.kernel,
    out_type=jax.ShapeDtypeStruct((NC, NS, RPT, D), jnp.float32),
    mesh=_MESH,
    scratch_types=[
        pltpu.VMEM((NCH, CH), jnp.int32),
        pltpu.VMEM((CH, D), jnp.float32),
        pltpu.VMEM((25, D), jnp.float32),
        pltpu.VMEM_SHARED((N, D), jnp.float32),
    ],
)
def _deg_kernel(src_hbm, degw_hbm, idx_v, ones_v, zrow_v, degacc):
    cid = lax.axis_index("c")
    sid = lax.axis_index("s")
    ewid = cid * NS + sid

    @pl.loop(0, CH)
    def _fill_ones(i):
        for c in range(D // 16):
            ones_v[i, pl.ds(c * 16, 16)] = jnp.ones((16,), jnp.float32)

    @pl.loop(0, 25)
    def _fill_zero(i):
        for c in range(D // 16):
            zrow_v[i, pl.ds(c * 16, 16)] = jnp.zeros((16,), jnp.float32)

    pltpu.sync_copy(src_hbm.at[ewid], idx_v)
    for j in range(RPT // 25):
        pltpu.sync_copy(zrow_v, degacc.at[pl.ds(sid * RPT + j * 25, 25)])
    plsc.subcore_barrier()

    @pl.loop(0, NCH)
    def _accum(i):
        pltpu.sync_copy(ones_v, degacc.at[idx_v.at[i]], add=True)

    plsc.subcore_barrier()
    pltpu.sync_copy(degacc.at[pl.ds(sid * RPT, RPT)],
                    degw_hbm.at[cid, sid])


@functools.partial(
    pl.kernel,
    out_type=jax.ShapeDtypeStruct((K, NC, NS, RPT, D), jnp.float32),
    mesh=_MESH,
    scratch_types=[
        pltpu.VMEM((NCH, CH), jnp.int32),
        pltpu.VMEM((NCH, CH), jnp.int32),
        pltpu.VMEM((CH, D), jnp.float32),
        pltpu.VMEM((25, D), jnp.float32),
        pltpu.VMEM_SHARED((N, D), jnp.float32),
        pltpu.SemaphoreType.DMA,
    ],
)
def _scatter_kernel(src_hbm, dst_hbm, hs_hbm, agg_hbm,
                    src_v, dst_v, rows_v, zb_v, acc, sem):
    cid = lax.axis_index("c")
    sid = lax.axis_index("s")
    ewid = cid * NS + sid
    zrows = 25

    @pl.loop(0, zrows)
    def _fill_zero(i):
        for c in range(D // 16):
            zb_v[i, pl.ds(c * 16, 16)] = jnp.zeros((16,), jnp.float32)

    pltpu.sync_copy(src_hbm.at[ewid], src_v)
    pltpu.sync_copy(dst_hbm.at[ewid], dst_v)

    for k in range(K):
        for j in range(RPT // 25):
            pltpu.sync_copy(zb_v, acc.at[pl.ds(sid * RPT + j * zrows, zrows)])
        plsc.subcore_barrier()

        @pl.loop(0, NCH)
        def _edge_chunk(i):
            pltpu.async_copy(hs_hbm.at[k].at[src_v.at[i]], rows_v, sem).wait()
            pltpu.sync_copy(rows_v, acc.at[dst_v.at[i]], add=True)

        plsc.subcore_barrier()
        pltpu.sync_copy(acc.at[pl.ds(sid * RPT, RPT)],
                        agg_hbm.at[k, cid, sid])


BN = 1000
NB = N // BN


def _mm_body(x_ref, w_ref, degw_ref, hs_ref):
    xb = x_ref[0]
    h = jnp.dot(xb, w_ref[...], preferred_element_type=jnp.float32)
    deg = degw_ref[0, :, 0] + degw_ref[1, :, 0] + 1.0
    dinv = lax.rsqrt(deg)
    hs_ref[0] = h * dinv[:, None]


def _mm(xt, W, degw):
    return pl.pallas_call(
        _mm_body,
        grid=(K, NB),
        in_specs=[
            pl.BlockSpec((1, BN, D), lambda k, i: (k, i, 0)),
            pl.BlockSpec((D, D), lambda k, i: (0, 0)),
            pl.BlockSpec((NC, BN, D), lambda k, i: (0, i, 0)),
        ],
        out_specs=pl.BlockSpec((1, BN, D), lambda k, i: (k, i, 0)),
        out_shape=jax.ShapeDtypeStruct((K, N, D), jnp.float32),
    )(xt, W, degw)


def _comb_body(agg_ref, hs_ref, degw_ref, o_ref):
    a = agg_ref[0, 0] + agg_ref[0, 1]
    deg = degw_ref[0, :, 0] + degw_ref[1, :, 0] + 1.0
    dinv = lax.rsqrt(deg)
    o_ref[0] = jnp.maximum((a + hs_ref[0]) * dinv[:, None], 0.0)


def _combine(agg, hs, degw):
    return pl.pallas_call(
        _comb_body,
        grid=(K, NB),
        in_specs=[
            pl.BlockSpec((1, NC, BN, D), lambda k, i: (k, 0, i, 0)),
            pl.BlockSpec((1, BN, D), lambda k, i: (k, i, 0)),
            pl.BlockSpec((NC, BN, D), lambda k, i: (0, i, 0)),
        ],
        out_specs=pl.BlockSpec((1, BN, D), lambda k, i: (k, i, 0)),
        out_shape=jax.ShapeDtypeStruct((K, N, D), jnp.float32),
    )(agg, hs, degw)


def kernel(x, W, edge_index):
    src = edge_index[0].reshape(NW, NCH, CH)
    dst = edge_index[1].reshape(NW, NCH, CH)
    degw = _deg_kernel(src).reshape(NC, N, D)
    xt = jnp.transpose(x, (1, 0, 2))
    hs = _mm(xt, W, degw)
    agg = _scatter_kernel(src, dst, hs).reshape(K, NC, N, D)
    out = _combine(agg, hs, degw)
    return jnp.transpose(out, (1, 0, 2))


# speedup vs baseline: 10.0855x; 1.0790x over previous
import functools
import jax
import jax.numpy as jnp
from jax import lax
from jax.experimental import pallas as pl
from jax.experimental.pallas import tpu as pltpu
import jax.experimental.pallas.tpu_sc as plsc

N = 10000
E = 320000
D = 128
K = 4
NC = 2
NS = 16
NW = NC * NS
EPW = E // NW
NCH = 200
CH = EPW // NCH
RPT = N // NS

_MESH = plsc.VectorSubcoreMesh(core_axis_name="c", subcore_axis_name="s",
                               num_cores=NC, num_subcores=NS)


@functools.partial(
    pl.kernel,
    out_type=jax.ShapeDtypeStruct((NC, NS, RPT, D), jnp.float32),
    mesh=_MESH,
    scratch_types=[
        pltpu.VMEM((NCH, CH), jnp.int32),
        pltpu.VMEM((CH, D), jnp.float32),
        pltpu.VMEM((25, D), jnp.float32),
        pltpu.VMEM_SHARED((N, D), jnp.float32),
    ],
)
def _deg_kernel(src_hbm, degw_hbm, idx_v, ones_v, zrow_v, degacc):
    cid = lax.axis_index("c")
    sid = lax.axis_index("s")
    ewid = cid * NS + sid

    @pl.loop(0, CH)
    def _fill_ones(i):
        for c in range(D // 16):
            ones_v[i, pl.ds(c * 16, 16)] = jnp.ones((16,), jnp.float32)

    @pl.loop(0, 25)
    def _fill_zero(i):
        for c in range(D // 16):
            zrow_v[i, pl.ds(c * 16, 16)] = jnp.zeros((16,), jnp.float32)

    pltpu.sync_copy(src_hbm.at[ewid], idx_v)
    for j in range(RPT // 25):
        pltpu.sync_copy(zrow_v, degacc.at[pl.ds(sid * RPT + j * 25, 25)])
    plsc.subcore_barrier()

    @pl.loop(0, NCH)
    def _accum(i):
        pltpu.sync_copy(ones_v, degacc.at[idx_v.at[i]], add=True)

    plsc.subcore_barrier()
    pltpu.sync_copy(degacc.at[pl.ds(sid * RPT, RPT)],
                    degw_hbm.at[cid, sid])


@functools.partial(
    pl.kernel,
    out_type=jax.ShapeDtypeStruct((K, NC, NS, RPT, D), jnp.float32),
    mesh=_MESH,
    scratch_types=[
        pltpu.VMEM((NCH // 2, CH), jnp.int32),
        pltpu.VMEM((NCH // 2, CH), jnp.int32),
        pltpu.VMEM((CH, D), jnp.float32),
        pltpu.VMEM((CH, D), jnp.float32),
        pltpu.VMEM_SHARED((N, D), jnp.float32),
        pltpu.SemaphoreType.DMA,
        pltpu.SemaphoreType.DMA,
    ],
)
def _scatter_kernel(src_hbm, dst_hbm, hs_hbm, agg_hbm,
                    src_v, dst_v, rows0_v, rows1_v, acc, sem0, sem1):
    cid = lax.axis_index("c")
    sid = lax.axis_index("s")
    ewid = cid * NS + sid

    for k in range(K):
        @pl.loop(0, CH)
        def _fill_zero(i):
            for c in range(D // 16):
                rows0_v[i, pl.ds(c * 16, 16)] = jnp.zeros((16,), jnp.float32)

        for j in range(RPT // CH):
            pltpu.sync_copy(rows0_v, acc.at[pl.ds(sid * RPT + j * CH, CH)])
        pltpu.sync_copy(rows0_v.at[pl.ds(0, RPT % CH)],
                        acc.at[pl.ds(sid * RPT + (RPT // CH) * CH, RPT % CH)])
        plsc.subcore_barrier()

        for h in range(2):
            pltpu.sync_copy(src_hbm.at[ewid, h], src_v)
            pltpu.sync_copy(dst_hbm.at[ewid, h], dst_v)

            @pl.loop(0, NCH // 2, step=2)
            def _edge_pair(i):
                d0 = pltpu.async_copy(hs_hbm.at[k].at[src_v.at[i]], rows0_v,
                                      sem0)
                d1 = pltpu.async_copy(hs_hbm.at[k].at[src_v.at[i + 1]],
                                      rows1_v, sem1)
                d0.wait()
                pltpu.sync_copy(rows0_v, acc.at[dst_v.at[i]], add=True)
                d1.wait()
                pltpu.sync_copy(rows1_v, acc.at[dst_v.at[i + 1]], add=True)

        plsc.subcore_barrier()
        pltpu.sync_copy(acc.at[pl.ds(sid * RPT, RPT)],
                        agg_hbm.at[k, cid, sid])


BN = 1000
NB = N // BN


def _mm_body(x_ref, w_ref, degw_ref, hs_ref):
    xb = x_ref[0]
    h = jnp.dot(xb, w_ref[...], preferred_element_type=jnp.float32)
    deg = degw_ref[0, :, 0] + degw_ref[1, :, 0] + 1.0
    dinv = lax.rsqrt(deg)
    hs_ref[0] = h * dinv[:, None]


def _mm(xt, W, degw):
    return pl.pallas_call(
        _mm_body,
        grid=(K, NB),
        in_specs=[
            pl.BlockSpec((1, BN, D), lambda k, i: (k, i, 0)),
            pl.BlockSpec((D, D), lambda k, i: (0, 0)),
            pl.BlockSpec((NC, BN, D), lambda k, i: (0, i, 0)),
        ],
        out_specs=pl.BlockSpec((1, BN, D), lambda k, i: (k, i, 0)),
        out_shape=jax.ShapeDtypeStruct((K, N, D), jnp.float32),
    )(xt, W, degw)


def _comb_body(agg_ref, hs_ref, degw_ref, o_ref):
    a = agg_ref[0, 0] + agg_ref[0, 1]
    deg = degw_ref[0, :, 0] + degw_ref[1, :, 0] + 1.0
    dinv = lax.rsqrt(deg)
    o_ref[0] = jnp.maximum((a + hs_ref[0]) * dinv[:, None], 0.0)


def _combine(agg, hs, degw):
    return pl.pallas_call(
        _comb_body,
        grid=(K, NB),
        in_specs=[
            pl.BlockSpec((1, NC, BN, D), lambda k, i: (k, 0, i, 0)),
            pl.BlockSpec((1, BN, D), lambda k, i: (k, i, 0)),
            pl.BlockSpec((NC, BN, D), lambda k, i: (0, i, 0)),
        ],
        out_specs=pl.BlockSpec((1, BN, D), lambda k, i: (k, i, 0)),
        out_shape=jax.ShapeDtypeStruct((K, N, D), jnp.float32),
    )(agg, hs, degw)


def kernel(x, W, edge_index):
    src = edge_index[0].reshape(NW, NCH, CH)
    dst = edge_index[1].reshape(NW, NCH, CH)
    src2 = edge_index[0].reshape(NW, 2, NCH // 2, CH)
    dst2 = edge_index[1].reshape(NW, 2, NCH // 2, CH)
    degw = _deg_kernel(src).reshape(NC, N, D)
    xt = jnp.transpose(x, (1, 0, 2))
    hs = _mm(xt, W, degw)
    agg = _scatter_kernel(src2, dst2, hs).reshape(K, NC, N, D)
    out = _combine(agg, hs, degw)
    return jnp.transpose(out, (1, 0, 2))


# speedup vs baseline: 10.3438x; 1.0256x over previous
import functools
import jax
import jax.numpy as jnp
from jax import lax
from jax.experimental import pallas as pl
from jax.experimental.pallas import tpu as pltpu
import jax.experimental.pallas.tpu_sc as plsc

N = 10000
E = 320000
D = 128
K = 4
NC = 2
NS = 16
NW = NC * NS
EPW = E // NW
NCH = 200
CH = EPW // NCH
RPT = N // NS

_MESH = plsc.VectorSubcoreMesh(core_axis_name="c", subcore_axis_name="s",
                               num_cores=NC, num_subcores=NS)


@functools.partial(
    pl.kernel,
    out_type=jax.ShapeDtypeStruct((NC, NS, RPT, D), jnp.float32),
    mesh=_MESH,
    scratch_types=[
        pltpu.VMEM((NCH, CH), jnp.int32),
        pltpu.VMEM((CH, D), jnp.float32),
        pltpu.VMEM((25, D), jnp.float32),
        pltpu.VMEM_SHARED((N, D), jnp.float32),
    ],
)
def _deg_kernel(src_hbm, degw_hbm, idx_v, ones_v, zrow_v, degacc):
    cid = lax.axis_index("c")
    sid = lax.axis_index("s")
    ewid = cid * NS + sid

    @pl.loop(0, CH)
    def _fill_ones(i):
        for c in range(D // 16):
            ones_v[i, pl.ds(c * 16, 16)] = jnp.ones((16,), jnp.float32)

    @pl.loop(0, 25)
    def _fill_zero(i):
        for c in range(D // 16):
            zrow_v[i, pl.ds(c * 16, 16)] = jnp.zeros((16,), jnp.float32)

    pltpu.sync_copy(src_hbm.at[ewid], idx_v)
    for j in range(RPT // 25):
        pltpu.sync_copy(zrow_v, degacc.at[pl.ds(sid * RPT + j * 25, 25)])
    plsc.subcore_barrier()

    @pl.loop(0, NCH)
    def _accum(i):
        pltpu.sync_copy(ones_v, degacc.at[idx_v.at[i]], add=True)

    plsc.subcore_barrier()
    pltpu.sync_copy(degacc.at[pl.ds(sid * RPT, RPT)],
                    degw_hbm.at[cid, sid])


@functools.partial(
    pl.kernel,
    out_type=jax.ShapeDtypeStruct((K, NC, NS, RPT, D), jnp.float32),
    mesh=_MESH,
    scratch_types=[
        pltpu.VMEM((NCH // 2, CH), jnp.int32),
        pltpu.VMEM((NCH // 2, CH), jnp.int32),
        pltpu.VMEM((CH, D), jnp.float32),
        pltpu.VMEM((CH, D), jnp.float32),
        pltpu.VMEM_SHARED((N, D), jnp.float32),
        pltpu.SemaphoreType.DMA,
        pltpu.SemaphoreType.DMA,
        pltpu.SemaphoreType.DMA,
        pltpu.SemaphoreType.DMA,
    ],
)
def _scatter_kernel(src_hbm, dst_hbm, hs_hbm, agg_hbm,
                    src_v, dst_v, rows0_v, rows1_v, acc, sem0, sem1,
                    sem2, sem3):
    cid = lax.axis_index("c")
    sid = lax.axis_index("s")
    ewid = cid * NS + sid

    for k in range(K):
        @pl.loop(0, CH)
        def _fill_zero(i):
            for c in range(D // 16):
                rows0_v[i, pl.ds(c * 16, 16)] = jnp.zeros((16,), jnp.float32)

        for j in range(RPT // CH):
            pltpu.sync_copy(rows0_v, acc.at[pl.ds(sid * RPT + j * CH, CH)])
        pltpu.sync_copy(rows0_v.at[pl.ds(0, RPT % CH)],
                        acc.at[pl.ds(sid * RPT + (RPT // CH) * CH, RPT % CH)])
        plsc.subcore_barrier()

        for h in range(2):
            pltpu.sync_copy(src_hbm.at[ewid, h], src_v)
            pltpu.sync_copy(dst_hbm.at[ewid, h], dst_v)

            @pl.loop(0, NCH // 2, step=2)
            def _edge_pair(i):
                d0 = pltpu.async_copy(hs_hbm.at[k].at[src_v.at[i]], rows0_v,
                                      sem0)
                d1 = pltpu.async_copy(hs_hbm.at[k].at[src_v.at[i + 1]],
                                      rows1_v, sem1)
                d0.wait()
                s0 = pltpu.async_copy(rows0_v, acc.at[dst_v.at[i]], sem2,
                                      add=True)
                d1.wait()
                s1 = pltpu.async_copy(rows1_v, acc.at[dst_v.at[i + 1]], sem3,
                                      add=True)
                s0.wait()
                s1.wait()

        plsc.subcore_barrier()
        pltpu.sync_copy(acc.at[pl.ds(sid * RPT, RPT)],
                        agg_hbm.at[k, cid, sid])


BN = 1000
NB = N // BN


def _mm_body(x_ref, w_ref, degw_ref, hs_ref):
    xb = x_ref[0]
    h = jnp.dot(xb, w_ref[...], preferred_element_type=jnp.float32)
    deg = degw_ref[0, :, 0] + degw_ref[1, :, 0] + 1.0
    dinv = lax.rsqrt(deg)
    hs_ref[0] = h * dinv[:, None]


def _mm(xt, W, degw):
    return pl.pallas_call(
        _mm_body,
        grid=(K, NB),
        in_specs=[
            pl.BlockSpec((1, BN, D), lambda k, i: (k, i, 0)),
            pl.BlockSpec((D, D), lambda k, i: (0, 0)),
            pl.BlockSpec((NC, BN, D), lambda k, i: (0, i, 0)),
        ],
        out_specs=pl.BlockSpec((1, BN, D), lambda k, i: (k, i, 0)),
        out_shape=jax.ShapeDtypeStruct((K, N, D), jnp.float32),
    )(xt, W, degw)


def _comb_body(agg_ref, hs_ref, degw_ref, o_ref):
    a = agg_ref[0, 0] + agg_ref[0, 1]
    deg = degw_ref[0, :, 0] + degw_ref[1, :, 0] + 1.0
    dinv = lax.rsqrt(deg)
    o_ref[0] = jnp.maximum((a + hs_ref[0]) * dinv[:, None], 0.0)


def _combine(agg, hs, degw):
    return pl.pallas_call(
        _comb_body,
        grid=(K, NB),
        in_specs=[
            pl.BlockSpec((1, NC, BN, D), lambda k, i: (k, 0, i, 0)),
            pl.BlockSpec((1, BN, D), lambda k, i: (k, i, 0)),
            pl.BlockSpec((NC, BN, D), lambda k, i: (0, i, 0)),
        ],
        out_specs=pl.BlockSpec((1, BN, D), lambda k, i: (k, i, 0)),
        out_shape=jax.ShapeDtypeStruct((K, N, D), jnp.float32),
    )(agg, hs, degw)


def kernel(x, W, edge_index):
    src = edge_index[0].reshape(NW, NCH, CH)
    dst = edge_index[1].reshape(NW, NCH, CH)
    src2 = edge_index[0].reshape(NW, 2, NCH // 2, CH)
    dst2 = edge_index[1].reshape(NW, 2, NCH // 2, CH)
    degw = _deg_kernel(src).reshape(NC, N, D)
    xt = jnp.transpose(x, (1, 0, 2))
    hs = _mm(xt, W, degw)
    agg = _scatter_kernel(src2, dst2, hs).reshape(K, NC, N, D)
    out = _combine(agg, hs, degw)
    return jnp.transpose(out, (1, 0, 2))


# speedup vs baseline: 10.9353x; 1.0572x over previous
import functools
import jax
import jax.numpy as jnp
from jax import lax
from jax.experimental import pallas as pl
from jax.experimental.pallas import tpu as pltpu
import jax.experimental.pallas.tpu_sc as plsc

N = 10000
E = 320000
D = 128
K = 4
NC = 2
NS = 16
NW = NC * NS
EPW = E // NW
NCH = 200
CH = EPW // NCH
RPT = N // NS

_MESH = plsc.VectorSubcoreMesh(core_axis_name="c", subcore_axis_name="s",
                               num_cores=NC, num_subcores=NS)


@functools.partial(
    pl.kernel,
    out_type=jax.ShapeDtypeStruct((NC, NS, RPT, D), jnp.float32),
    mesh=_MESH,
    scratch_types=[
        pltpu.VMEM((NCH, CH), jnp.int32),
        pltpu.VMEM((CH, D), jnp.float32),
        pltpu.VMEM((25, D), jnp.float32),
        pltpu.VMEM_SHARED((N, D), jnp.float32),
    ],
)
def _deg_kernel(src_hbm, degw_hbm, idx_v, ones_v, zrow_v, degacc):
    cid = lax.axis_index("c")
    sid = lax.axis_index("s")
    ewid = cid * NS + sid

    @pl.loop(0, CH)
    def _fill_ones(i):
        for c in range(D // 16):
            ones_v[i, pl.ds(c * 16, 16)] = jnp.ones((16,), jnp.float32)

    @pl.loop(0, 25)
    def _fill_zero(i):
        for c in range(D // 16):
            zrow_v[i, pl.ds(c * 16, 16)] = jnp.zeros((16,), jnp.float32)

    pltpu.sync_copy(src_hbm.at[ewid], idx_v)
    for j in range(RPT // 25):
        pltpu.sync_copy(zrow_v, degacc.at[pl.ds(sid * RPT + j * 25, 25)])
    plsc.subcore_barrier()

    @pl.loop(0, NCH)
    def _accum(i):
        pltpu.sync_copy(ones_v, degacc.at[idx_v.at[i]], add=True)

    plsc.subcore_barrier()
    pltpu.sync_copy(degacc.at[pl.ds(sid * RPT, RPT)],
                    degw_hbm.at[cid, sid])


NSEC = 10
SCH = NCH // NSEC
NBUF = 4


@functools.partial(
    pl.kernel,
    out_type=jax.ShapeDtypeStruct((K, NC, NS, RPT, D), jnp.float32),
    mesh=_MESH,
    scratch_types=[
        pltpu.VMEM((SCH, CH), jnp.int32),
        pltpu.VMEM((SCH, CH), jnp.int32),
        [pltpu.VMEM((CH, D), jnp.float32)] * NBUF,
        pltpu.VMEM_SHARED((N, D), jnp.float32),
        [pltpu.SemaphoreType.DMA] * NBUF,
        [pltpu.SemaphoreType.DMA] * NBUF,
    ],
)
def _scatter_kernel(src_hbm, dst_hbm, hs_hbm, agg_hbm,
                    src_v, dst_v, rows, acc, gsems, ssems):
    cid = lax.axis_index("c")
    sid = lax.axis_index("s")
    ewid = cid * NS + sid

    for k in range(K):
        @pl.loop(0, CH)
        def _fill_zero(i):
            for c in range(D // 16):
                rows[0][i, pl.ds(c * 16, 16)] = jnp.zeros((16,), jnp.float32)

        for j in range(RPT // CH):
            pltpu.sync_copy(rows[0], acc.at[pl.ds(sid * RPT + j * CH, CH)])
        pltpu.sync_copy(rows[0].at[pl.ds(0, RPT % CH)],
                        acc.at[pl.ds(sid * RPT + (RPT // CH) * CH, RPT % CH)])
        plsc.subcore_barrier()

        for h in range(NSEC):
            pltpu.sync_copy(src_hbm.at[ewid, h], src_v)
            pltpu.sync_copy(dst_hbm.at[ewid, h], dst_v)

            @pl.loop(0, SCH, step=NBUF)
            def _edge_block(i):
                gd = [pltpu.async_copy(hs_hbm.at[k].at[src_v.at[i + b]],
                                       rows[b], gsems[b])
                      for b in range(NBUF)]
                sd = []
                for b in range(NBUF):
                    gd[b].wait()
                    sd.append(pltpu.async_copy(rows[b],
                                               acc.at[dst_v.at[i + b]],
                                               ssems[b], add=True))
                for b in range(NBUF):
                    sd[b].wait()

        plsc.subcore_barrier()
        pltpu.sync_copy(acc.at[pl.ds(sid * RPT, RPT)],
                        agg_hbm.at[k, cid, sid])


BN = 1000
NB = N // BN


def _mm_body(x_ref, w_ref, degw_ref, hs_ref):
    xb = x_ref[0]
    h = jnp.dot(xb, w_ref[...], preferred_element_type=jnp.float32)
    deg = degw_ref[0, :, 0] + degw_ref[1, :, 0] + 1.0
    dinv = lax.rsqrt(deg)
    hs_ref[0] = h * dinv[:, None]


def _mm(xt, W, degw):
    return pl.pallas_call(
        _mm_body,
        grid=(K, NB),
        in_specs=[
            pl.BlockSpec((1, BN, D), lambda k, i: (k, i, 0)),
            pl.BlockSpec((D, D), lambda k, i: (0, 0)),
            pl.BlockSpec((NC, BN, D), lambda k, i: (0, i, 0)),
        ],
        out_specs=pl.BlockSpec((1, BN, D), lambda k, i: (k, i, 0)),
        out_shape=jax.ShapeDtypeStruct((K, N, D), jnp.float32),
    )(xt, W, degw)


def _comb_body(agg_ref, hs_ref, degw_ref, o_ref):
    a = agg_ref[0, 0] + agg_ref[0, 1]
    deg = degw_ref[0, :, 0] + degw_ref[1, :, 0] + 1.0
    dinv = lax.rsqrt(deg)
    o_ref[0] = jnp.maximum((a + hs_ref[0]) * dinv[:, None], 0.0)


def _combine(agg, hs, degw):
    return pl.pallas_call(
        _comb_body,
        grid=(K, NB),
        in_specs=[
            pl.BlockSpec((1, NC, BN, D), lambda k, i: (k, 0, i, 0)),
            pl.BlockSpec((1, BN, D), lambda k, i: (k, i, 0)),
            pl.BlockSpec((NC, BN, D), lambda k, i: (0, i, 0)),
        ],
        out_specs=pl.BlockSpec((1, BN, D), lambda k, i: (k, i, 0)),
        out_shape=jax.ShapeDtypeStruct((K, N, D), jnp.float32),
    )(agg, hs, degw)


def kernel(x, W, edge_index):
    src = edge_index[0].reshape(NW, NCH, CH)
    dst = edge_index[1].reshape(NW, NCH, CH)
    src2 = edge_index[0].reshape(NW, NSEC, SCH, CH)
    dst2 = edge_index[1].reshape(NW, NSEC, SCH, CH)
    degw = _deg_kernel(src).reshape(NC, N, D)
    xt = jnp.transpose(x, (1, 0, 2))
    hs = _mm(xt, W, degw)
    agg = _scatter_kernel(src2, dst2, hs).reshape(K, NC, N, D)
    out = _combine(agg, hs, degw)
    return jnp.transpose(out, (1, 0, 2))
